# Initial kernel scaffold; baseline (speedup 1.0000x reference)
import jax
import jax.numpy as jnp
from jax.experimental import pallas as pl


def kernel(x, embeddings):
    raise NotImplementedError("write your pallas kernel here")


# speedup vs baseline: 1.6812x; 1.6812x over previous
import functools

import jax
import jax.numpy as jnp
from jax import lax
from jax.experimental import pallas as pl
from jax.experimental.pallas import tpu as pltpu
from jax.experimental.pallas import tpu_sc as plsc

_N_EMBEDS = 1024
_EMBED_DIM = 64
_BETA = 0.25

_ROWS_PER_BLOCK = 1024

_NC = 2
_NS = 16
_NW = _NC * _NS
_IDX_CHUNK = 128


def _tc_body(x_ref, emb_ref, idx_ref, embt_ref, loss_ref):
    pid = pl.program_id(0)
    nblocks = pl.num_programs(0)

    xb = x_ref[...]
    emb = emb_ref[...]
    sim = jnp.dot(xb, emb, preferred_element_type=jnp.float32)
    x2 = jnp.sum(xb * xb, axis=1, keepdims=True)
    e2 = jnp.sum(emb * emb, axis=0, keepdims=True)
    dists = x2 + e2 - 2.0 * sim

    minv = jnp.min(dists, axis=1, keepdims=True)
    cols = lax.broadcasted_iota(jnp.int32, dists.shape, 1)
    idx = jnp.min(jnp.where(dists == minv, cols, _N_EMBEDS), axis=1,
                  keepdims=True)
    idx_ref[...] = idx

    @pl.when(pid == 0)
    def _init():
        embt_ref[...] = emb.T
        loss_ref[0, 0] = 0.0

    loss_ref[0, 0] += jnp.sum(minv)

    @pl.when(pid == nblocks - 1)
    def _finish():
        total = jnp.float32(32 * 1024 * _EMBED_DIM)
        loss_ref[0, 0] = loss_ref[0, 0] * ((1.0 + _BETA) / total)


def _tc_stage(flat, embeddings):
    n_rows = flat.shape[0]
    nblocks = n_rows // _ROWS_PER_BLOCK
    return pl.pallas_call(
        _tc_body,
        grid=(nblocks,),
        in_specs=[
            pl.BlockSpec((_ROWS_PER_BLOCK, _EMBED_DIM), lambda i: (i, 0)),
            pl.BlockSpec((_EMBED_DIM, _N_EMBEDS), lambda i: (0, 0)),
        ],
        out_specs=[
            pl.BlockSpec((_ROWS_PER_BLOCK, 1), lambda i: (i, 0)),
            pl.BlockSpec((_N_EMBEDS, _EMBED_DIM), lambda i: (0, 0)),
            pl.BlockSpec(memory_space=pltpu.SMEM, block_shape=(1, 1),
                         index_map=lambda i: (0, 0)),
        ],
        out_shape=[
            jax.ShapeDtypeStruct((n_rows, 1), jnp.int32),
            jax.ShapeDtypeStruct((_N_EMBEDS, _EMBED_DIM), jnp.float32),
            jax.ShapeDtypeStruct((1, 1), jnp.float32),
        ],
    )(flat, embeddings)


def _make_sc_gather(n_rows):
    rows_per_w = n_rows // _NW
    chunks = rows_per_w // _IDX_CHUNK
    mesh = plsc.VectorSubcoreMesh(core_axis_name="c", subcore_axis_name="s")

    @functools.partial(
        pl.kernel,
        mesh=mesh,
        out_type=jax.ShapeDtypeStruct((n_rows, _EMBED_DIM), jnp.float32),
        scratch_types=[
            pltpu.VMEM((chunks, _IDX_CHUNK), jnp.int32),
            pltpu.VMEM((rows_per_w, _EMBED_DIM), jnp.float32),
            pltpu.SemaphoreType.DMA,
        ],
        compiler_params=pltpu.CompilerParams(use_tc_tiling_on_sc=False),
    )
    def _sc_gather(table_hbm, idx_hbm, out_hbm, idx_v, rows_v, sem):
        wid = lax.axis_index("s") * _NC + lax.axis_index("c")
        pltpu.sync_copy(idx_hbm.at[wid], idx_v)
        copies = []
        for j in range(chunks):
            copies.append(
                pltpu.async_copy(
                    table_hbm.at[idx_v.at[j]],
                    rows_v.at[pl.ds(j * _IDX_CHUNK, _IDX_CHUNK)],
                    sem,
                ))
        for c in copies:
            c.wait()
        pltpu.sync_copy(rows_v, out_hbm.at[pl.ds(wid * rows_per_w, rows_per_w)])

    return _sc_gather


def kernel(x, embeddings):
    in_shape = x.shape
    flat = x.reshape(-1, _EMBED_DIM)
    n_rows = flat.shape[0]

    idx, table, loss = _tc_stage(flat, embeddings)
    idx3 = idx.reshape(_NW, (n_rows // _NW) // _IDX_CHUNK, _IDX_CHUNK)

    qtised = _make_sc_gather(n_rows)(table, idx3)
    return (qtised.reshape(in_shape), loss.reshape(()))


# speedup vs baseline: 1.7440x; 1.0373x over previous
import functools

import jax
import jax.numpy as jnp
from jax import lax
from jax.experimental import pallas as pl
from jax.experimental.pallas import tpu as pltpu
from jax.experimental.pallas import tpu_sc as plsc

_N_EMBEDS = 1024
_EMBED_DIM = 64
_PAD_DIM = 128
_BETA = 0.25

_ROWS_PER_BLOCK = 1024

_NC = 2
_NS = 16
_NW = _NC * _NS
_IDX_CHUNK = 128


def _tc_body(x_ref, emb_ref, idx_ref, embt_ref, loss_ref):
    pid = pl.program_id(0)
    nblocks = pl.num_programs(0)

    xb = x_ref[...]
    emb = emb_ref[...]
    sim = jnp.dot(xb, emb, preferred_element_type=jnp.float32)
    x2 = jnp.sum(xb * xb, axis=1, keepdims=True)
    e2 = jnp.sum(emb * emb, axis=0, keepdims=True)
    dists = x2 + e2 - 2.0 * sim

    minv = jnp.min(dists, axis=1, keepdims=True)
    cols = lax.broadcasted_iota(jnp.int32, dists.shape, 1)
    idx = jnp.min(jnp.where(dists == minv, cols, _N_EMBEDS), axis=1)
    idx_ref[...] = idx.reshape(_ROWS_PER_BLOCK // _IDX_CHUNK, _IDX_CHUNK)

    @pl.when(pid == 0)
    def _init():
        embt_ref[...] = emb.T
        loss_ref[0, 0] = 0.0

    loss_ref[0, 0] += jnp.sum(minv)

    @pl.when(pid == nblocks - 1)
    def _finish():
        total = jnp.float32(32 * 1024 * _EMBED_DIM)
        loss_ref[0, 0] = loss_ref[0, 0] * ((1.0 + _BETA) / total)


def _tc_stage(flat, embeddings):
    n_rows = flat.shape[0]
    nblocks = n_rows // _ROWS_PER_BLOCK
    idx_rows_blk = _ROWS_PER_BLOCK // _IDX_CHUNK
    return pl.pallas_call(
        _tc_body,
        grid=(nblocks,),
        in_specs=[
            pl.BlockSpec((_ROWS_PER_BLOCK, _EMBED_DIM), lambda i: (i, 0)),
            pl.BlockSpec((_EMBED_DIM, _N_EMBEDS), lambda i: (0, 0)),
        ],
        out_specs=[
            pl.BlockSpec((idx_rows_blk, _IDX_CHUNK), lambda i: (i, 0)),
            pl.BlockSpec((_N_EMBEDS, _EMBED_DIM), lambda i: (0, 0)),
            pl.BlockSpec(memory_space=pltpu.SMEM, block_shape=(1, 1),
                         index_map=lambda i: (0, 0)),
        ],
        out_shape=[
            jax.ShapeDtypeStruct((n_rows // _IDX_CHUNK, _IDX_CHUNK), jnp.int32),
            jax.ShapeDtypeStruct((_N_EMBEDS, _EMBED_DIM), jnp.float32),
            jax.ShapeDtypeStruct((1, 1), jnp.float32),
        ],
    )(flat, embeddings)


def _make_sc_gather(n_rows):
    rows_per_w = n_rows // _NW
    chunks = rows_per_w // _IDX_CHUNK
    idx_rows_w = rows_per_w // _IDX_CHUNK
    mesh = plsc.VectorSubcoreMesh(core_axis_name="c", subcore_axis_name="s")

    @functools.partial(
        pl.kernel,
        mesh=mesh,
        out_type=jax.ShapeDtypeStruct((n_rows, _EMBED_DIM), jnp.float32),
        scratch_types=[
            pltpu.VMEM((idx_rows_w, _IDX_CHUNK), jnp.int32),
            pltpu.VMEM((2, _IDX_CHUNK, _EMBED_DIM), jnp.float32),
            pltpu.SemaphoreType.DMA,
            pltpu.SemaphoreType.DMA,
            pltpu.SemaphoreType.DMA,
        ],
        compiler_params=pltpu.CompilerParams(use_tc_tiling_on_sc=False),
    )
    def _sc_gather(table_hbm, idx_hbm, out_hbm, idx_v, rows_v, gsem, wsem0,
                   wsem1):
        wid = lax.axis_index("s") * _NC + lax.axis_index("c")
        base = wid * rows_per_w
        pltpu.sync_copy(idx_hbm.at[pl.ds(wid * idx_rows_w, idx_rows_w)], idx_v)

        wsems = (wsem0, wsem1)
        gathers = [None] * chunks
        writes = [None] * chunks
        gathers[0] = pltpu.async_copy(
            table_hbm.at[idx_v.at[0]], rows_v.at[0], gsem)
        for j in range(chunks):
            b = j % 2
            gathers[j].wait()
            if j + 1 < chunks:
                if j >= 1:
                    writes[j - 1].wait()
                gathers[j + 1] = pltpu.async_copy(
                    table_hbm.at[idx_v.at[j + 1]], rows_v.at[(j + 1) % 2], gsem)
            writes[j] = pltpu.async_copy(
                rows_v.at[b],
                out_hbm.at[pl.ds(base + j * _IDX_CHUNK, _IDX_CHUNK)],
                wsems[b])
        writes[chunks - 2].wait()
        writes[chunks - 1].wait()

    return _sc_gather


def kernel(x, embeddings):
    in_shape = x.shape
    flat = x.reshape(-1, _EMBED_DIM)
    n_rows = flat.shape[0]

    idx, table, loss = _tc_stage(flat, embeddings)
    qtised = _make_sc_gather(n_rows)(table, idx)
    return (qtised.reshape(in_shape), loss.reshape(()))


# speedup vs baseline: 1.8944x; 1.0863x over previous
import functools

import jax
import jax.numpy as jnp
from jax import lax
from jax.experimental import pallas as pl
from jax.experimental.pallas import tpu as pltpu
from jax.experimental.pallas import tpu_sc as plsc

_N_EMBEDS = 1024
_EMBED_DIM = 64
_BETA = 0.25

_B = 32
_T = 1024
_LANES = 128
_SUB = 8
_NC = 2
_NS = 16
_NW = _NC * _NS


def _tc_body(xt_ref, emb_ref, idx_ref, embt_ref, loss_ref):
    pid = pl.program_id(0)
    nblocks = pl.num_programs(0)

    xb = xt_ref[0]
    emb = emb_ref[...]
    sim = lax.dot_general(emb, xb, (((0,), (0,)), ((), ())),
                          preferred_element_type=jnp.float32)
    x2 = jnp.sum(xb * xb, axis=0, keepdims=True)
    e2 = jnp.sum(emb * emb, axis=0, keepdims=True)
    dists = x2 + e2.reshape(_N_EMBEDS, 1) - 2.0 * sim

    minv = jnp.min(dists, axis=0, keepdims=True)
    rows = lax.broadcasted_iota(jnp.int32, dists.shape, 0)
    idx = jnp.min(jnp.where(dists == minv, rows, _N_EMBEDS), axis=0)
    idx_ref[...] = idx.reshape(_T // _LANES, _LANES)

    @pl.when(pid == 0)
    def _init():
        embt_ref[...] = emb.T
        loss_ref[0, 0] = 0.0

    loss_ref[0, 0] += jnp.sum(minv)

    @pl.when(pid == nblocks - 1)
    def _finish():
        total = jnp.float32(_B * _T * _EMBED_DIM)
        loss_ref[0, 0] = loss_ref[0, 0] * ((1.0 + _BETA) / total)


def _tc_stage(xt, embeddings):
    idx_rows_blk = _T // _LANES
    return pl.pallas_call(
        _tc_body,
        grid=(_B,),
        in_specs=[
            pl.BlockSpec((1, _EMBED_DIM, _T), lambda i: (i, 0, 0)),
            pl.BlockSpec((_EMBED_DIM, _N_EMBEDS), lambda i: (0, 0)),
        ],
        out_specs=[
            pl.BlockSpec((idx_rows_blk, _LANES), lambda i: (i, 0)),
            pl.BlockSpec((_N_EMBEDS, _EMBED_DIM), lambda i: (0, 0)),
            pl.BlockSpec(memory_space=pltpu.SMEM, block_shape=(1, 1),
                         index_map=lambda i: (0, 0)),
        ],
        out_shape=[
            jax.ShapeDtypeStruct((_B * idx_rows_blk, _LANES), jnp.int32),
            jax.ShapeDtypeStruct((_N_EMBEDS, _EMBED_DIM), jnp.float32),
            jax.ShapeDtypeStruct((1, 1), jnp.float32),
        ],
    )(xt, embeddings)


def _make_sc_gather():
    tchunks = _T // _LANES
    groups = _LANES // 16
    mesh = plsc.VectorSubcoreMesh(core_axis_name="c", subcore_axis_name="s")

    @functools.partial(
        pl.kernel,
        mesh=mesh,
        out_type=jax.ShapeDtypeStruct(
            (_B, _SUB, tchunks, _SUB, _LANES), jnp.float32),
        scratch_types=[
            pltpu.VMEM((_N_EMBEDS * _EMBED_DIM,), jnp.float32),
            pltpu.VMEM((tchunks, _LANES), jnp.int32),
            pltpu.VMEM((2, _EMBED_DIM, _LANES), jnp.float32),
            pltpu.SemaphoreType.DMA,
            pltpu.SemaphoreType.DMA,
            pltpu.SemaphoreType.DMA,
        ],
        compiler_params=pltpu.CompilerParams(use_tc_tiling_on_sc=False,
                                             needs_layout_passes=False),
    )
    def _sc_gather(table_hbm, idx_hbm, out_hbm, table_v, idx_v, bufs, tsem,
                   wsem0, wsem1):
        b = lax.axis_index("s") * _NC + lax.axis_index("c")
        pltpu.sync_copy(idx_hbm.at[pl.ds(b * tchunks, tchunks)], idx_v)
        pltpu.sync_copy(table_hbm, table_v)

        def assemble(tc, buf):
            for g in range(groups):
                tok_idx = idx_v[tc, pl.ds(g * 16, 16)]
                base = tok_idx * _EMBED_DIM
                for e in range(_EMBED_DIM):
                    buf[e, pl.ds(g * 16, 16)] = plsc.load_gather(
                        table_v, [base + e])

        def fire(tc, buf, wsem):
            for ts in range(_SUB):
                pltpu.async_copy(buf.at[pl.ds(ts * _SUB, _SUB)],
                                 out_hbm.at[b, ts, tc], wsem)

        def drain(tc, buf, wsem):
            for ts in range(_SUB):
                pltpu.make_async_copy(buf.at[pl.ds(ts * _SUB, _SUB)],
                                      out_hbm.at[b, ts, tc], wsem).wait()

        def body(i, carry):
            tc0 = 2 * i
            tc1 = tc0 + 1

            @pl.when(i > 0)
            def _():
                drain(tc0 - 2, bufs.at[0], wsem0)

            assemble(tc0, bufs.at[0])
            fire(tc0, bufs.at[0], wsem0)

            @pl.when(i > 0)
            def _():
                drain(tc1 - 2, bufs.at[1], wsem1)

            assemble(tc1, bufs.at[1])
            fire(tc1, bufs.at[1], wsem1)
            return carry

        lax.fori_loop(0, tchunks // 2, body, 0)
        drain(tchunks - 2, bufs.at[0], wsem0)
        drain(tchunks - 1, bufs.at[1], wsem1)

    return _sc_gather


def kernel(x, embeddings):
    xt = jnp.swapaxes(x, 1, 2)
    idx, table, loss = _tc_stage(xt, embeddings)

    out5 = _make_sc_gather()(table.reshape(_N_EMBEDS * _EMBED_DIM), idx)
    qtised = out5.transpose(0, 2, 4, 1, 3).reshape(_B, _T, _EMBED_DIM)
    return (qtised, loss.reshape(()))


# speedup vs baseline: 2.3815x; 1.2571x over previous
import functools

import jax
import jax.numpy as jnp
from jax import lax
from jax.experimental import pallas as pl
from jax.experimental.pallas import tpu as pltpu
from jax.experimental.pallas import tpu_sc as plsc

_N_EMBEDS = 1024
_EMBED_DIM = 64
_BETA = 0.25

_B = 32
_T = 1024
_LANES = 128
_SUB = 8
_NC = 2
_NS = 16
_NW = _NC * _NS


def _tc_body(xt_ref, emb_ref, idx_ref, embt_ref, loss_ref):
    pid = pl.program_id(0)
    nblocks = pl.num_programs(0)

    xb = xt_ref[0]
    emb = emb_ref[...]
    sim = lax.dot_general(emb, xb, (((0,), (0,)), ((), ())),
                          preferred_element_type=jnp.float32)
    x2 = jnp.sum(xb * xb, axis=0, keepdims=True)
    e2 = jnp.sum(emb * emb, axis=0, keepdims=True)
    dists = x2 + e2.reshape(_N_EMBEDS, 1) - 2.0 * sim

    minv = jnp.min(dists, axis=0, keepdims=True)
    rows = lax.broadcasted_iota(jnp.int32, dists.shape, 0)
    idx = jnp.min(jnp.where(dists == minv, rows, _N_EMBEDS), axis=0)
    idx_ref[...] = idx.reshape(_T // _LANES, _LANES)

    @pl.when(pid == 0)
    def _init():
        embt_ref[...] = emb.T
        loss_ref[0, 0] = 0.0

    loss_ref[0, 0] += jnp.sum(minv)

    @pl.when(pid == nblocks - 1)
    def _finish():
        total = jnp.float32(_B * _T * _EMBED_DIM)
        loss_ref[0, 0] = loss_ref[0, 0] * ((1.0 + _BETA) / total)


def _tc_stage(xt, embeddings):
    idx_rows_blk = _T // _LANES
    return pl.pallas_call(
        _tc_body,
        grid=(_B,),
        in_specs=[
            pl.BlockSpec((1, _EMBED_DIM, _T), lambda i: (i, 0, 0)),
            pl.BlockSpec((_EMBED_DIM, _N_EMBEDS), lambda i: (0, 0)),
        ],
        out_specs=[
            pl.BlockSpec((idx_rows_blk, _LANES), lambda i: (i, 0)),
            pl.BlockSpec((_N_EMBEDS, _EMBED_DIM), lambda i: (0, 0)),
            pl.BlockSpec(memory_space=pltpu.SMEM, block_shape=(1, 1),
                         index_map=lambda i: (0, 0)),
        ],
        out_shape=[
            jax.ShapeDtypeStruct((_B * idx_rows_blk, _LANES), jnp.int32),
            jax.ShapeDtypeStruct((_N_EMBEDS, _EMBED_DIM), jnp.float32),
            jax.ShapeDtypeStruct((1, 1), jnp.float32),
        ],
    )(xt, embeddings)


def _make_sc_gather():
    tchunks = _T // _LANES
    _PITCH = _LANES + 1
    mesh = plsc.VectorSubcoreMesh(core_axis_name="c", subcore_axis_name="s")

    @functools.partial(
        pl.kernel,
        mesh=mesh,
        out_type=jax.ShapeDtypeStruct(
            (_B, _SUB, tchunks, _SUB, _LANES), jnp.float32),
        scratch_types=[
            pltpu.VMEM((_N_EMBEDS * _EMBED_DIM,), jnp.float32),
            pltpu.VMEM((tchunks, _LANES), jnp.int32),
            pltpu.VMEM((_EMBED_DIM, _PITCH), jnp.float32),
            pltpu.VMEM((_EMBED_DIM, _PITCH), jnp.float32),
            pltpu.SemaphoreType.DMA,
            pltpu.SemaphoreType.DMA,
            pltpu.SemaphoreType.DMA,
        ],
        compiler_params=pltpu.CompilerParams(use_tc_tiling_on_sc=False,
                                             needs_layout_passes=False),
    )
    def _sc_gather(table_hbm, idx_hbm, out_hbm, table_v, idx_v, buf0, buf1,
                   tsem, wsem0, wsem1):
        b = lax.axis_index("s") * _NC + lax.axis_index("c")
        pltpu.sync_copy(idx_hbm.at[pl.ds(b * tchunks, tchunks)], idx_v)
        pltpu.sync_copy(table_hbm, table_v)

        lane = lax.iota(jnp.int32, 16)
        rows16 = [lane + 16 * k for k in range(_EMBED_DIM // 16)]

        def assemble(tc, buf):
            for g in range(_LANES // 16):
                offs = idx_v[tc, pl.ds(g * 16, 16)] * _EMBED_DIM
                for j in range(16):
                    t = g * 16 + j
                    off = offs[j]
                    col = jnp.full((16,), t, jnp.int32)
                    for k in range(_EMBED_DIM // 16):
                        v = table_v[pl.ds(off + 16 * k, 16)]
                        plsc.store_scatter(buf, [rows16[k], col], v)

        def fire(tc, buf, wsem):
            for ts in range(_SUB):
                pltpu.async_copy(
                    buf.at[pl.ds(ts * _SUB, _SUB), pl.ds(0, _LANES)],
                    out_hbm.at[b, ts, tc], wsem)

        def drain(tc, buf, wsem):
            for ts in range(_SUB):
                pltpu.make_async_copy(
                    buf.at[pl.ds(ts * _SUB, _SUB), pl.ds(0, _LANES)],
                    out_hbm.at[b, ts, tc], wsem).wait()

        def body(i, carry):
            tc0 = 2 * i
            tc1 = tc0 + 1

            @pl.when(i > 0)
            def _():
                drain(tc0 - 2, buf0, wsem0)

            assemble(tc0, buf0)
            fire(tc0, buf0, wsem0)

            @pl.when(i > 0)
            def _():
                drain(tc1 - 2, buf1, wsem1)

            assemble(tc1, buf1)
            fire(tc1, buf1, wsem1)
            return carry

        lax.fori_loop(0, tchunks // 2, body, 0)
        drain(tchunks - 2, buf0, wsem0)
        drain(tchunks - 1, buf1, wsem1)

    return _sc_gather


def kernel(x, embeddings):
    xt = jnp.swapaxes(x, 1, 2)
    idx, table, loss = _tc_stage(xt, embeddings)

    out5 = _make_sc_gather()(table.reshape(_N_EMBEDS * _EMBED_DIM), idx)
    qtised = out5.transpose(0, 2, 4, 1, 3).reshape(_B, _T, _EMBED_DIM)
    return (qtised, loss.reshape(()))
